# baseline (speedup 1.0000x reference)
import functools

import jax
import jax.numpy as jnp
from jax import lax
from jax.experimental import pallas as pl
from jax.experimental.pallas import tpu as pltpu
from jax.experimental.pallas import tpu_sc as plsc

_D = 64
_NC, _NS = 2, 16
_NW = _NC * _NS
_CB = 512


@functools.partial(jax.jit, static_argnums=(2,))
def _sc_gather(table, idx, B):
    b_per_w = B // _NW
    n_chunks = b_per_w // _CB
    mesh = plsc.VectorSubcoreMesh(core_axis_name="c", subcore_axis_name="s")

    @functools.partial(
        pl.kernel,
        mesh=mesh,
        out_type=jax.ShapeDtypeStruct((B, _D), jnp.float32),
        scratch_types=[
            pltpu.VMEM((_CB,), jnp.int32),
            pltpu.VMEM((_CB, _D), jnp.float32),
            pltpu.SemaphoreType.DMA,
        ],
        compiler_params=pltpu.CompilerParams(use_tc_tiling_on_sc=False),
    )
    def k(table_hbm, idx_hbm, out_hbm, idx_v, rows_v, sem):
        wid = lax.axis_index("s") * _NC + lax.axis_index("c")
        base = wid * b_per_w

        def body(i, carry):
            off = base + i * _CB
            pltpu.sync_copy(idx_hbm.at[pl.ds(off, _CB)], idx_v)
            pltpu.async_copy(table_hbm.at[idx_v], rows_v, sem).wait()
            pltpu.sync_copy(rows_v, out_hbm.at[pl.ds(off, _CB)])
            return carry

        lax.fori_loop(0, n_chunks, body, 0)

    return k(table, idx)


def kernel(inputs, table):
    n, s = inputs.shape
    B = n * s
    idx = inputs.reshape(B).astype(jnp.int32)
    out = _sc_gather(table, idx, B)
    return out.reshape(n, s, _D)


# speedup vs baseline: 1.0421x; 1.0421x over previous
import functools

import jax
import jax.numpy as jnp
from jax import lax
from jax.experimental import pallas as pl
from jax.experimental.pallas import tpu as pltpu
from jax.experimental.pallas import tpu_sc as plsc

_D = 64
_NC, _NS = 2, 16
_NW = _NC * _NS
_CB = 512
_NBUF = 2


@functools.partial(jax.jit, static_argnums=(2,))
def _sc_gather(table, idx, B):
    b_per_w = B // _NW
    n_chunks = b_per_w // _CB
    assert n_chunks % 2 == 0 and n_chunks >= 4
    mesh = plsc.VectorSubcoreMesh(core_axis_name="c", subcore_axis_name="s")

    @functools.partial(
        pl.kernel,
        mesh=mesh,
        out_type=jax.ShapeDtypeStruct((B, _D), jnp.float32),
        scratch_types=[
            pltpu.VMEM((b_per_w,), jnp.int32),
            pltpu.VMEM((_NBUF, _CB, _D), jnp.float32),
            pltpu.SemaphoreType.DMA,
            pltpu.SemaphoreType.DMA,
            pltpu.SemaphoreType.DMA,
            pltpu.SemaphoreType.DMA,
        ],
        compiler_params=pltpu.CompilerParams(use_tc_tiling_on_sc=False),
    )
    def k(table_hbm, idx_hbm, out_hbm, idx_v, rows_v, g0, g1, w0, w1):
        wid = lax.axis_index("s") * _NC + lax.axis_index("c")
        base = wid * b_per_w
        gsem = (g0, g1)
        wsem = (w0, w1)

        pltpu.sync_copy(idx_hbm.at[pl.ds(base, b_per_w)], idx_v)

        def gather(i, b):
            return pltpu.make_async_copy(
                table_hbm.at[idx_v.at[pl.ds(i * _CB, _CB)]],
                rows_v.at[b], gsem[b])

        def writeback(i, b):
            return pltpu.make_async_copy(
                rows_v.at[b], out_hbm.at[pl.ds(base + i * _CB, _CB)], wsem[b])

        for b in range(_NBUF):
            gather(b, b).start()

        def body(g, carry):
            for b in range(_NBUF):
                i = _NBUF * g + b
                gather(i, b).wait()
                writeback(i, b).start()
                writeback(i, b).wait()
                gather(i + _NBUF, b).start()
            return carry

        lax.fori_loop(0, n_chunks // _NBUF - 1, body, 0)

        for b in range(_NBUF):
            i = n_chunks - _NBUF + b
            gather(i, b).wait()
            writeback(i, b).start()
            writeback(i, b).wait()

    return k(table, idx)


def kernel(inputs, table):
    n, s = inputs.shape
    B = n * s
    idx = inputs.reshape(B).astype(jnp.int32)
    out = _sc_gather(table, idx, B)
    return out.reshape(n, s, _D)
